# baseline (speedup 1.0000x reference)
import functools

import jax
import jax.numpy as jnp
from jax import lax
from jax.experimental import pallas as pl
from jax.experimental.pallas import tpu as pltpu
from jax.experimental.pallas import tpu_sc as plsc

_D = 128
_CHUNK = 128


@functools.lru_cache(maxsize=None)
def _build(n_idx, n_rows):
    info = plsc.get_sparse_core_info()
    nc, ns = info.num_cores, info.num_subcores
    nw = nc * ns
    assert n_rows % (nw * _CHUNK) == 0
    n_chunks = n_idx
    NBUF = 2
    AHEAD = 1
    assert n_chunks % NBUF == 0 and AHEAD < NBUF
    mesh = plsc.VectorSubcoreMesh(core_axis_name="c", subcore_axis_name="s")

    @functools.partial(
        pl.kernel,
        mesh=mesh,
        out_type=jax.ShapeDtypeStruct((n_idx * n_rows, _D), jnp.float32),
        scratch_types=[
            pltpu.VMEM((n_idx, _CHUNK), jnp.int32),
        ]
        + [pltpu.VMEM((_CHUNK, _D), jnp.float32) for _ in range(NBUF)]
        + [pltpu.SemaphoreType.DMA for _ in range(2 * NBUF)],
        compiler_params=pltpu.CompilerParams(
            use_tc_tiling_on_sc=True, skip_device_barrier=True
        ),
    )
    def gather_kernel(xt_hbm, table_hbm, out_hbm, idx_v, *rest):
        bufs = rest[:NBUF]
        gsem = rest[NBUF : 2 * NBUF]
        ssem = rest[2 * NBUF :]
        wid = lax.axis_index("s") * nc + lax.axis_index("c")
        col0 = wid * _CHUNK
        pltpu.sync_copy(xt_hbm.at[:, pl.ds(col0, _CHUNK)], idx_v)

        def gather(j, b):
            return pltpu.make_async_copy(
                table_hbm.at[idx_v.at[j]], bufs[b], gsem[b]
            )

        def store(j, b):
            return pltpu.make_async_copy(
                bufs[b],
                out_hbm.at[pl.ds(j * n_rows + col0, _CHUNK)],
                ssem[b],
            )

        for b in range(AHEAD):
            gather(b, b).start()

        def outer(g, _):
            for b in range(NBUF):
                j = g * NBUF + b
                bf = (b + AHEAD) % NBUF

                @pl.when(j + AHEAD < n_chunks)
                def _issue():
                    @pl.when(j + AHEAD >= NBUF)
                    def _drain():
                        store(j + AHEAD - NBUF, bf).wait()

                    gather(j + AHEAD, bf).start()

                gather(j, b).wait()
                store(j, b).start()
            return 0

        lax.fori_loop(0, n_chunks // NBUF, outer, 0)
        for b in range(NBUF):
            store(n_chunks - NBUF + b, b).wait()

    return gather_kernel


def kernel(x, weight):
    n_rows, n_idx = x.shape
    out = _build(n_idx, n_rows)(x.T.astype(jnp.int32), weight)
    return out.reshape(n_idx, n_rows, _D).transpose(1, 0, 2)


# speedup vs baseline: 1.0238x; 1.0238x over previous
import functools

import jax
import jax.numpy as jnp
from jax import lax
from jax.experimental import pallas as pl
from jax.experimental.pallas import tpu as pltpu
from jax.experimental.pallas import tpu_sc as plsc

_D = 128
_CHUNK = 128


@functools.lru_cache(maxsize=None)
def _build(n_idx, n_rows):
    info = plsc.get_sparse_core_info()
    nc, ns = info.num_cores, info.num_subcores
    nw = nc * ns
    assert n_rows % (nw * _CHUNK) == 0
    n_chunks = n_idx
    NBUF = 5
    AHEAD = 4
    assert n_chunks % NBUF == 0 and AHEAD < NBUF
    mesh = plsc.VectorSubcoreMesh(core_axis_name="c", subcore_axis_name="s")

    @functools.partial(
        pl.kernel,
        mesh=mesh,
        out_type=jax.ShapeDtypeStruct((n_idx * n_rows, _D), jnp.float32),
        scratch_types=[
            pltpu.VMEM((n_idx, _CHUNK), jnp.int32),
        ]
        + [pltpu.VMEM((_CHUNK, _D), jnp.float32) for _ in range(NBUF)]
        + [pltpu.SemaphoreType.DMA for _ in range(2 * NBUF)],
        compiler_params=pltpu.CompilerParams(
            use_tc_tiling_on_sc=True, skip_device_barrier=True
        ),
    )
    def gather_kernel(xt_hbm, table_hbm, out_hbm, idx_v, *rest):
        bufs = rest[:NBUF]
        gsem = rest[NBUF : 2 * NBUF]
        ssem = rest[2 * NBUF :]
        wid = lax.axis_index("s") * nc + lax.axis_index("c")
        col0 = wid * _CHUNK
        pltpu.sync_copy(xt_hbm.at[:, pl.ds(col0, _CHUNK)], idx_v)

        def gather(j, b):
            return pltpu.make_async_copy(
                table_hbm.at[idx_v.at[j]], bufs[b], gsem[b]
            )

        def store(j, b):
            return pltpu.make_async_copy(
                bufs[b],
                out_hbm.at[pl.ds(j * n_rows + col0, _CHUNK)],
                ssem[b],
            )

        for b in range(AHEAD):
            gather(b, b).start()

        def outer(g, _):
            for b in range(NBUF):
                j = g * NBUF + b
                bf = (b + AHEAD) % NBUF

                @pl.when(j + AHEAD < n_chunks)
                def _issue():
                    @pl.when(j + AHEAD >= NBUF)
                    def _drain():
                        store(j + AHEAD - NBUF, bf).wait()

                    gather(j + AHEAD, bf).start()

                gather(j, b).wait()
                store(j, b).start()
            return 0

        lax.fori_loop(0, n_chunks // NBUF, outer, 0)
        for b in range(NBUF):
            store(n_chunks - NBUF + b, b).wait()

    return gather_kernel


def kernel(x, weight):
    n_rows, n_idx = x.shape
    out = _build(n_idx, n_rows)(x.T.astype(jnp.int32), weight)
    return out.reshape(n_idx, n_rows, _D).transpose(1, 0, 2)
